# Initial kernel scaffold; baseline (speedup 1.0000x reference)
import jax
import jax.numpy as jnp
from jax.experimental import pallas as pl


def kernel(x_target, source_feats, source_importance):
    raise NotImplementedError("write your pallas kernel here")


# speedup vs baseline: 2.3762x; 2.3762x over previous
import functools

import jax
import jax.numpy as jnp
from jax import lax
from jax.experimental import pallas as pl
from jax.experimental.pallas import tpu as pltpu

N, T, C = 512, 64, 2
E_S, F_DIM = 65536, 16
CAND_TOPK = 16
RETR_TOPK = 8
W_IMP = 0.1
N_LAGS = 16
M = N * CAND_TOPK

NEG = -1e30


def _p1_body(x0_ref, x1_ref, idx_ref, feats_ref):
    x0 = x0_ref[:]
    x1 = x1_ref[:]
    s = jnp.sum(x0, axis=1, keepdims=True) + jnp.sum(x1, axis=1, keepdims=True)
    m = s / float(T * C)
    v = (jnp.sum((x0 - m) ** 2, axis=1, keepdims=True)
         + jnp.sum((x1 - m) ** 2, axis=1, keepdims=True)) / float(T * C)
    sd = jnp.sqrt(v) + 1e-8
    z0 = (x0 - m) / sd
    z1 = (x1 - m) / sd
    dn = (((1,), (1,)), ((), ()))
    corr = (lax.dot_general(z0, z0, dn, preferred_element_type=jnp.float32)
            + lax.dot_general(z1, z1, dn, preferred_element_type=jnp.float32)
            ) / float(T * C)
    iota_r = lax.broadcasted_iota(jnp.int32, (N, N), 0)
    iota_c = lax.broadcasted_iota(jnp.int32, (N, N), 1)
    corr = jnp.where(iota_r == iota_c, corr - 1e9, corr)

    cols = []
    cwork = corr
    for _ in range(CAND_TOPK):
        mx = jnp.max(cwork, axis=1, keepdims=True)
        sel = cwork == mx
        ai = jnp.min(jnp.where(sel, iota_c, N), axis=1, keepdims=True)
        cols.append(ai)
        cwork = jnp.where(iota_c == ai, NEG, cwork)
    idx = jnp.concatenate(cols, axis=1)
    idx_ref[:] = idx

    m0 = jnp.mean(x0, axis=1, keepdims=True)
    sd0 = jnp.sqrt(jnp.mean((x0 - m0) ** 2, axis=1, keepdims=True)) + 1e-8
    xn0 = (x0 - m0) / sd0
    m1 = jnp.mean(x1, axis=1, keepdims=True)
    sd1 = jnp.sqrt(jnp.mean((x1 - m1) ** 2, axis=1, keepdims=True)) + 1e-8
    xn1 = (x1 - m1) / sd1

    iota_j = lax.broadcasted_iota(jnp.int32, (N, CAND_TOPK, N), 2)
    oh = (idx[:, :, None] == iota_j).astype(jnp.float32)
    for l in range(N_LAGS):
        a0 = xn0[:, : T - l]
        b0 = xn0[:, l:]
        a1 = xn1[:, : T - l]
        b1 = xn1[:, l:]
        g = (lax.dot_general(a0, b0, dn, preferred_element_type=jnp.float32, precision=lax.Precision.HIGHEST)
             + lax.dot_general(a1, b1, dn, preferred_element_type=jnp.float32, precision=lax.Precision.HIGHEST)
             ) / float((T - l) * C)
        feats_ref[:, :, l] = jnp.sum(oh * g[:, None, :], axis=2)


def _p1(x0, x1):
    return pl.pallas_call(
        _p1_body,
        out_shape=(
            jax.ShapeDtypeStruct((N, CAND_TOPK), jnp.int32),
            jax.ShapeDtypeStruct((N, CAND_TOPK, N_LAGS), jnp.float32),
        ),
    )(x0, x1)


P2_BQ = 256
P2_BK = 8192
P2_NQ = M // P2_BQ
P2_NK = E_S // P2_BK


def _p2_body(feats_ref, kT_ref, imp_ref, vals_ref, pimp_ref, vals_s, pimp_s):
    kk = pl.program_id(1)

    @pl.when(kk == 0)
    def _init():
        vals_s[:] = jnp.full((P2_BQ, RETR_TOPK), NEG, jnp.float32)
        pimp_s[:] = jnp.zeros((P2_BQ, RETR_TOPK), jnp.float32)

    f = feats_ref[:]
    qn = f / (jnp.sqrt(jnp.sum(f * f, axis=1, keepdims=True)) + 1e-8)
    kT = kT_ref[:]
    kn = kT / (jnp.sqrt(jnp.sum(kT * kT, axis=0, keepdims=True)) + 1e-8)
    sim = lax.dot_general(qn, kn, (((1,), (0,)), ((), ())),
                          preferred_element_type=jnp.float32)
    imp = imp_ref[0]
    score = sim + W_IMP * imp

    iota_c = lax.broadcasted_iota(jnp.int32, (P2_BQ, P2_BK), 1)
    bvals, bimps = [], []
    for _ in range(RETR_TOPK):
        mx = jnp.max(score, axis=1, keepdims=True)
        sel = score == mx
        ai = jnp.min(jnp.where(sel, iota_c, P2_BK), axis=1, keepdims=True)
        hit = iota_c == ai
        pi = jnp.max(jnp.where(hit, imp, NEG), axis=1, keepdims=True)
        score = jnp.where(hit, NEG, score)
        bvals.append(mx)
        bimps.append(pi)

    cand_v = jnp.concatenate([vals_s[:]] + bvals, axis=1)
    cand_i = jnp.concatenate([pimp_s[:]] + bimps, axis=1)
    iota16 = lax.broadcasted_iota(jnp.int32, (P2_BQ, 2 * RETR_TOPK), 1)
    nv, ni = [], []
    for _ in range(RETR_TOPK):
        mx = jnp.max(cand_v, axis=1, keepdims=True)
        sel = cand_v == mx
        ai = jnp.min(jnp.where(sel, iota16, 2 * RETR_TOPK), axis=1, keepdims=True)
        hit = iota16 == ai
        pi = jnp.max(jnp.where(hit, cand_i, NEG), axis=1, keepdims=True)
        cand_v = jnp.where(hit, NEG, cand_v)
        nv.append(mx)
        ni.append(pi)
    vals_s[:] = jnp.concatenate(nv, axis=1)
    pimp_s[:] = jnp.concatenate(ni, axis=1)

    @pl.when(kk == P2_NK - 1)
    def _out():
        vals_ref[:] = vals_s[:]
        pimp_ref[:] = pimp_s[:]


def _p2(feats2d, kT, imp3d):
    return pl.pallas_call(
        _p2_body,
        grid=(P2_NQ, P2_NK),
        in_specs=[
            pl.BlockSpec((P2_BQ, F_DIM), lambda q, k: (q, 0)),
            pl.BlockSpec((F_DIM, P2_BK), lambda q, k: (0, k)),
            pl.BlockSpec((1, 1, P2_BK), lambda q, k: (k, 0, 0)),
        ],
        out_specs=(
            pl.BlockSpec((P2_BQ, RETR_TOPK), lambda q, k: (q, 0)),
            pl.BlockSpec((P2_BQ, RETR_TOPK), lambda q, k: (q, 0)),
        ),
        out_shape=(
            jax.ShapeDtypeStruct((M, RETR_TOPK), jnp.float32),
            jax.ShapeDtypeStruct((M, RETR_TOPK), jnp.float32),
        ),
        scratch_shapes=[
            pltpu.VMEM((P2_BQ, RETR_TOPK), jnp.float32),
            pltpu.VMEM((P2_BQ, RETR_TOPK), jnp.float32),
        ],
    )(feats2d, kT, imp3d)


def _p3_body(vals_ref, pimp_ref, idx_ref, a_ref, loss_ref):
    s = vals_ref[:]
    mx = jnp.max(s, axis=2, keepdims=True)
    e = jnp.exp(s - mx)
    alpha = e / jnp.sum(e, axis=2, keepdims=True)
    es = jnp.sum(alpha * pimp_ref[:], axis=2)

    iota_j = lax.broadcasted_iota(jnp.int32, (N, CAND_TOPK, N), 2)
    oh = (idx_ref[:][:, :, None] == iota_j).astype(jnp.float32)
    a = jnp.sum(oh * es[:, :, None], axis=1)

    maskb = jnp.abs(a) > 0
    logits = jnp.where(maskb, a, -1e9)
    rm = jnp.max(logits, axis=1, keepdims=True)
    ex = jnp.exp(logits - rm)
    sm = ex / jnp.sum(ex, axis=1, keepdims=True)
    out = sm * maskb.astype(jnp.float32)
    den = jnp.clip(jnp.sum(out, axis=1, keepdims=True), 1e-12, None)
    out = out / den
    a_ref[:] = out
    tot = jnp.sum(jnp.abs(out), axis=1, keepdims=True)
    loss_ref[:] = jnp.sum(tot, axis=0, keepdims=True) / float(N * N)


def _p3(vals3d, pimp3d, idx):
    return pl.pallas_call(
        _p3_body,
        out_shape=(
            jax.ShapeDtypeStruct((N, N), jnp.float32),
            jax.ShapeDtypeStruct((1, 1), jnp.float32),
        ),
    )(vals3d, pimp3d, idx)


@jax.jit
def kernel(x_target, source_feats, source_importance):
    x0 = x_target[:, :, 0]
    x1 = x_target[:, :, 1]
    idx, feats = _p1(x0, x1)
    feats2d = feats.reshape(M, F_DIM)
    kT = source_feats.T
    imp3d = source_importance.reshape(P2_NK, 1, P2_BK)
    vals, pimp = _p2(feats2d, kT, imp3d)
    a, loss = _p3(vals.reshape(N, CAND_TOPK, RETR_TOPK),
                  pimp.reshape(N, CAND_TOPK, RETR_TOPK), idx)
    return a, loss.reshape(())
